# Initial kernel scaffold; baseline (speedup 1.0000x reference)
import jax
import jax.numpy as jnp
from jax.experimental import pallas as pl


def kernel(feature, task_ids, W, b):
    raise NotImplementedError("write your pallas kernel here")


# speedup vs baseline: 1.6033x; 1.6033x over previous
import functools

import jax
import jax.numpy as jnp
from jax.experimental import pallas as pl

NUM_TASKS = 8
NUM_TOKENS = 4096
INPUT_DIM = 1024
NUM_CLASSES = 512

TOK_BLK = 1024


def _mm_body(x_ref, t_ref, w_ref, b_ref, o_ref):
    e = pl.program_id(1)

    @pl.when(e == 0)
    def _():
        o_ref[...] = jnp.zeros_like(o_ref)

    x = x_ref[...]
    w = w_ref[0]
    y = jax.lax.dot_general(x, w, (((1,), (1,)), ((), ())),
                            preferred_element_type=jnp.float32)
    mask = (t_ref[...] == e).astype(jnp.float32)[:, :1]
    o_ref[...] += mask * (y + b_ref[0, 0][None, :])


def kernel(feature, task_ids, W, b):
    t2d = jnp.broadcast_to(task_ids.astype(jnp.int32)[:, None],
                           (NUM_TOKENS, 128))
    grid = (NUM_TOKENS // TOK_BLK, NUM_TASKS)
    out = pl.pallas_call(
        _mm_body,
        grid=grid,
        in_specs=[
            pl.BlockSpec((TOK_BLK, INPUT_DIM), lambda j, e: (j, 0)),
            pl.BlockSpec((TOK_BLK, 128), lambda j, e: (j, 0)),
            pl.BlockSpec((1, NUM_CLASSES, INPUT_DIM), lambda j, e: (e, 0, 0)),
            pl.BlockSpec((1, 1, NUM_CLASSES), lambda j, e: (e, 0, 0)),
        ],
        out_specs=pl.BlockSpec((TOK_BLK, NUM_CLASSES), lambda j, e: (j, 0)),
        out_shape=jax.ShapeDtypeStruct((NUM_TOKENS, NUM_CLASSES), jnp.float32),
    )(feature, t2d, W, b.reshape(NUM_TASKS, 1, NUM_CLASSES))
    return out
